# baseline (speedup 1.0000x reference)
import functools

import jax
import jax.numpy as jnp
from jax import lax
from jax.experimental import pallas as pl
from jax.experimental.pallas import tpu as pltpu
from jax.experimental.pallas import tpu_sc as plsc

N_FACTORS = 100
BATCH = 16384
LANES = 16
NC = 2
NS = 16
NW = NC * NS
B_PER_W = BATCH // NW
CH = 128
NCH = B_PER_W // CH
CGP = CH // LANES
NSLOTS = 2
NFULL = 6
TAIL_OFF = N_FACTORS - LANES
WPACK = (2 * (NFULL + 1) + 1) * LANES


def _body(uid_hbm, cid_hbm, uf_hbm, cf_hbm, wpack_hbm,
          out_hbm,
          uidx_v, cidx_v, ub0, ub1, cb0, cb1, wp_v, rot_v,
          out_v, su0, su1, sc0, sc1):
    ubufs = (ub0, ub1)
    cbufs = (cb0, cb1)
    usems = (su0, su1)
    csems = (sc0, sc1)

    wid = lax.axis_index("s") * NC + lax.axis_index("c")
    base = wid * B_PER_W

    pltpu.sync_copy(uid_hbm.at[pl.ds(base, B_PER_W)], uidx_v)
    pltpu.sync_copy(cid_hbm.at[pl.ds(base, B_PER_W)], cidx_v)
    pltpu.sync_copy(wpack_hbm, wp_v)

    def start(k, slot):
        ubuf, cbuf = ubufs[slot], cbufs[slot]

        def issue(g, _):
            goff = pl.multiple_of(k * CH + g * LANES, LANES)
            iu = uidx_v[pl.ds(goff, LANES)]
            ic = cidx_v[pl.ds(goff, LANES)]
            for j in range(LANES):
                r = g * LANES + j
                pltpu.async_copy(uf_hbm.at[iu[j]], ubuf.at[r], usems[slot])
                pltpu.async_copy(cf_hbm.at[ic[j]], cbuf.at[r], csems[slot])
            return 0

        lax.fori_loop(0, CGP, issue, 0)

    def drain(slot):
        pltpu.make_async_copy(uf_hbm.at[pl.ds(0, CH)], ubufs[slot],
                              usems[slot]).wait()
        pltpu.make_async_copy(cf_hbm.at[pl.ds(0, CH)], cbufs[slot],
                              csems[slot]).wait()

    wu = [wp_v[pl.ds(i * LANES, LANES)] for i in range(NFULL + 1)]
    wc = [wp_v[pl.ds((NFULL + 1 + i) * LANES, LANES)]
          for i in range(NFULL + 1)]
    bb16 = wp_v[pl.ds(2 * (NFULL + 1) * LANES, LANES)]
    lane = lax.iota(jnp.int32, LANES)

    for k in range(NSLOTS):
        start(k, k % NSLOTS)

    for k in range(NCH):
        slot = k % NSLOTS
        drain(slot)
        ubuf = ubufs[slot]
        cbuf = cbufs[slot]

        def gbody(g, _, ubuf=ubuf, cbuf=cbuf, k=k):
            res = jnp.zeros((LANES,), jnp.float32)
            for j in range(LANES):
                r = g * LANES + j
                acc = bb16
                for t in range(NFULL):
                    acc = acc + ubuf[r, pl.ds(t * LANES, LANES)] * wu[t]
                    acc = acc + cbuf[r, pl.ds(t * LANES, LANES)] * wc[t]
                acc = acc + ubuf[r, pl.ds(TAIL_OFF, LANES)] * wu[NFULL]
                acc = acc + cbuf[r, pl.ds(TAIL_OFF, LANES)] * wc[NFULL]
                rb = 2 * LANES * j
                for d in (8, 4, 2, 1):
                    rot_v[pl.ds(rb, LANES)] = acc
                    rot_v[pl.ds(rb + LANES, LANES)] = acc
                    acc = acc + rot_v[pl.ds(rb + d, LANES)]
                res = jnp.where(lane == j, acc, res)
            off = pl.multiple_of(k * CH + g * LANES, LANES)
            out_v[pl.ds(off, LANES)] = res
            return 0

        lax.fori_loop(0, CGP, gbody, 0)

        nxt = k + NSLOTS
        if nxt < NCH:
            start(nxt, slot)

    pltpu.sync_copy(out_v, out_hbm.at[pl.ds(base, B_PER_W)])


@functools.partial(
    pl.kernel,
    mesh=plsc.VectorSubcoreMesh(core_axis_name="c", subcore_axis_name="s"),
    out_type=jax.ShapeDtypeStruct((BATCH,), jnp.float32),
    scratch_types=[
        pltpu.VMEM((B_PER_W,), jnp.int32),
        pltpu.VMEM((B_PER_W,), jnp.int32),
        pltpu.VMEM((CH, N_FACTORS), jnp.float32),
        pltpu.VMEM((CH, N_FACTORS), jnp.float32),
        pltpu.VMEM((CH, N_FACTORS), jnp.float32),
        pltpu.VMEM((CH, N_FACTORS), jnp.float32),
        pltpu.VMEM((WPACK,), jnp.float32),
        pltpu.VMEM((2 * LANES * LANES,), jnp.float32),
        pltpu.VMEM((B_PER_W,), jnp.float32),
        pltpu.SemaphoreType.DMA,
        pltpu.SemaphoreType.DMA,
        pltpu.SemaphoreType.DMA,
        pltpu.SemaphoreType.DMA,
    ],
)
def _sc_recommend(uid, cid, uf, cf, wpack, out,
                  uidx_v, cidx_v, ub0, ub1, cb0, cb1, wp_v,
                  rot_v, out_v, su0, su1, sc0, sc1):
    _body(uid, cid, uf, cf, wpack, out,
          uidx_v, cidx_v, ub0, ub1, cb0, cb1, wp_v, rot_v,
          out_v, su0, su1, sc0, sc1)


def _pack_weights(fc_w, fc_b):
    w = fc_w[:, 0]
    wu_full = w[:NFULL * LANES]
    tail_feats = jnp.arange(TAIL_OFF, TAIL_OFF + LANES)
    wu_tail = jnp.where(tail_feats >= NFULL * LANES,
                        w[TAIL_OFF:TAIL_OFF + LANES], 0.0)
    wc_off = N_FACTORS
    wc_full = w[wc_off:wc_off + NFULL * LANES]
    wc_tail = jnp.where(tail_feats >= NFULL * LANES,
                        w[wc_off + TAIL_OFF:wc_off + TAIL_OFF + LANES], 0.0)
    bb16 = jnp.broadcast_to(fc_b / LANES, (LANES,)).astype(jnp.float32)
    return jnp.concatenate([wu_full, wu_tail, wc_full, wc_tail, bb16])


def kernel(user_ids, course_ids, user_factors, course_factors, fc_w, fc_b):
    wpack = _pack_weights(fc_w, fc_b)
    return _sc_recommend(user_ids.astype(jnp.int32),
                         course_ids.astype(jnp.int32),
                         user_factors, course_factors, wpack)


# speedup vs baseline: 1.9835x; 1.9835x over previous
import functools

import jax
import jax.numpy as jnp
from jax import lax
from jax.experimental import pallas as pl
from jax.experimental.pallas import tpu as pltpu
from jax.experimental.pallas import tpu_sc as plsc

N_FACTORS = 100
BATCH = 16384
LANES = 16
NC = 2
NS = 16
NW = NC * NS
B_PER_W = BATCH // NW
CH = 128
NCH = B_PER_W // CH
CGP = CH // LANES
NSLOTS = 2
MV_NB = 8192


def _mv_body(x_ref, w_ref, o_ref):
    o_ref[...] = jnp.dot(w_ref[...], x_ref[...],
                         preferred_element_type=jnp.float32)


def _tc_project(xt, w_row):
    f, n = xt.shape
    grid = (pl.cdiv(n, MV_NB),)
    return pl.pallas_call(
        _mv_body,
        grid=grid,
        in_specs=[
            pl.BlockSpec((f, MV_NB), lambda i: (0, i)),
            pl.BlockSpec((1, f), lambda i: (0, 0)),
        ],
        out_specs=pl.BlockSpec((1, MV_NB), lambda i: (0, i)),
        out_shape=jax.ShapeDtypeStruct((1, n), jnp.float32),
    )(xt, w_row)


def _sc_body(uid_hbm, cid_hbm, pu_hbm, pc_hbm, bb_hbm,
             out_hbm,
             uidx_v, cidx_v, urow_v, crow_v, ub0, ub1, cb0, cb1,
             bb_v, rot_v, out_v,
             su0, su1, sc0, sc1):
    ubufs = (ub0, ub1)
    cbufs = (cb0, cb1)
    usems = (su0, su1)
    csems = (sc0, sc1)

    wid = lax.axis_index("s") * NC + lax.axis_index("c")
    base = wid * B_PER_W

    pltpu.sync_copy(uid_hbm.at[pl.ds(base, B_PER_W)], uidx_v)
    pltpu.sync_copy(cid_hbm.at[pl.ds(base, B_PER_W)], cidx_v)
    pltpu.sync_copy(bb_hbm, bb_v)

    for q in range(B_PER_W // LANES):
        off = q * LANES
        urow_v[pl.ds(off, LANES)] = lax.shift_right_logical(
            uidx_v[pl.ds(off, LANES)], 7)
        crow_v[pl.ds(off, LANES)] = lax.shift_right_logical(
            cidx_v[pl.ds(off, LANES)], 7)

    def start(k, slot):
        hu = pltpu.async_copy(pu_hbm.at[urow_v.at[pl.ds(k * CH, CH)]],
                              ubufs[slot], usems[slot])
        hc = pltpu.async_copy(pc_hbm.at[crow_v.at[pl.ds(k * CH, CH)]],
                              cbufs[slot], csems[slot])
        return hu, hc

    bvec = bb_v[:]
    lane = lax.iota(jnp.int32, LANES)

    handles = [None] * NCH
    for k in range(NSLOTS):
        handles[k] = start(k, k % NSLOTS)

    for k in range(NCH):
        slot = k % NSLOTS
        hu, hc = handles[k]
        hu.wait()
        hc.wait()
        ubuf = ubufs[slot]
        cbuf = cbufs[slot]

        def gbody(g, _, ubuf=ubuf, cbuf=cbuf, k=k):
            goff = pl.multiple_of(k * CH + g * LANES, LANES)
            iu = uidx_v[pl.ds(goff, LANES)]
            ic = cidx_v[pl.ds(goff, LANES)]
            res = bvec
            for j in range(LANES):
                r = g * LANES + j

                def pick(buf, idx_vec, rb):
                    w = idx_vec[j] & 127
                    coff = pl.multiple_of(w & 112, LANES)
                    v = buf[r, pl.ds(coff, LANES)]
                    rot_v[pl.ds(rb, LANES)] = v
                    rot_v[pl.ds(rb + LANES, LANES)] = v
                    return rot_v[pl.ds(rb + (((w & 15) - j + LANES) & 15),
                                       LANES)]

                tu = pick(ubuf, iu, 4 * LANES * j)
                tc_ = pick(cbuf, ic, 4 * LANES * j + 2 * LANES)
                res = jnp.where(lane == j, res + tu + tc_, res)
            out_v[pl.ds(goff, LANES)] = res
            return 0

        lax.fori_loop(0, CGP, gbody, 0)

        nxt = k + NSLOTS
        if nxt < NCH:
            handles[nxt] = start(nxt, slot)

    pltpu.sync_copy(out_v, out_hbm.at[pl.ds(base, B_PER_W)])


def _make_sc_gather(nru, nrc):
    return functools.partial(
        pl.kernel,
        mesh=plsc.VectorSubcoreMesh(core_axis_name="c", subcore_axis_name="s"),
        out_type=jax.ShapeDtypeStruct((BATCH,), jnp.float32),
        scratch_types=[
            pltpu.VMEM((B_PER_W,), jnp.int32),
            pltpu.VMEM((B_PER_W,), jnp.int32),
            pltpu.VMEM((B_PER_W,), jnp.int32),
            pltpu.VMEM((B_PER_W,), jnp.int32),
            pltpu.VMEM((CH, 128), jnp.float32),
            pltpu.VMEM((CH, 128), jnp.float32),
            pltpu.VMEM((CH, 128), jnp.float32),
            pltpu.VMEM((CH, 128), jnp.float32),
            pltpu.VMEM((LANES,), jnp.float32),
            pltpu.VMEM((4 * LANES * LANES,), jnp.float32),
            pltpu.VMEM((B_PER_W,), jnp.float32),
            pltpu.SemaphoreType.DMA,
            pltpu.SemaphoreType.DMA,
            pltpu.SemaphoreType.DMA,
            pltpu.SemaphoreType.DMA,
        ],
    )(_sc_body)


_SC_GATHER = None


def kernel(user_ids, course_ids, user_factors, course_factors, fc_w, fc_b):
    global _SC_GATHER
    nu = user_factors.shape[0]
    ncr = course_factors.shape[0]

    wu_row = fc_w[:N_FACTORS].T
    wc_row = fc_w[N_FACTORS:].T
    p_u = _tc_project(user_factors.T, wu_row)[0]
    p_c = _tc_project(course_factors.T, wc_row)[0]

    nru = (nu + 127) // 128
    nrc = (ncr + 127) // 128
    pu2 = jnp.pad(p_u, (0, nru * 128 - nu)).reshape(nru, 128)
    pc2 = jnp.pad(p_c, (0, nrc * 128 - ncr)).reshape(nrc, 128)
    bb16 = jnp.broadcast_to(fc_b, (LANES,)).astype(jnp.float32)

    if _SC_GATHER is None:
        _SC_GATHER = _make_sc_gather(nru, nrc)
    return _SC_GATHER(user_ids.astype(jnp.int32),
                      course_ids.astype(jnp.int32),
                      pu2, pc2, bb16)


# speedup vs baseline: 2.7772x; 1.4001x over previous
import functools

import jax
import jax.numpy as jnp
from jax import lax
from jax.experimental import pallas as pl
from jax.experimental.pallas import tpu as pltpu
from jax.experimental.pallas import tpu_sc as plsc

N_FACTORS = 100
BATCH = 16384
LANES = 16
NC = 2
NS = 16
NW = NC * NS
B_PER_W = BATCH // NW
CH = 128
NCH = B_PER_W // CH
CGP = CH // LANES
NSLOTS = 2
MV_NB = 16384


def _mv_body(x_ref, w_ref, o_ref):
    o_ref[...] = jnp.dot(w_ref[...], x_ref[...],
                         preferred_element_type=jnp.float32)[0]


def _tc_project(xt, w_row):
    f, n = xt.shape
    grid = (pl.cdiv(n, MV_NB),)
    return pl.pallas_call(
        _mv_body,
        grid=grid,
        in_specs=[
            pl.BlockSpec((f, MV_NB), lambda i: (0, i)),
            pl.BlockSpec((1, f), lambda i: (0, 0)),
        ],
        out_specs=pl.BlockSpec((MV_NB,), lambda i: (i,)),
        out_shape=jax.ShapeDtypeStruct((n,), jnp.float32),
    )(xt, w_row)


def _sc_body(uid_hbm, cid_hbm, pu_hbm, pc_hbm, bb_hbm,
             out_hbm,
             uidx_v, cidx_v, urow_v, crow_v, ub0, ub1, cb0, cb1,
             bb_v, rot_v, out_v,
             su0, su1, sc0, sc1):
    ubufs = (ub0, ub1)
    cbufs = (cb0, cb1)
    usems = (su0, su1)
    csems = (sc0, sc1)

    wid = lax.axis_index("s") * NC + lax.axis_index("c")
    base = wid * B_PER_W

    pltpu.sync_copy(uid_hbm.at[pl.ds(base, B_PER_W)], uidx_v)
    pltpu.sync_copy(cid_hbm.at[pl.ds(base, B_PER_W)], cidx_v)
    pltpu.sync_copy(bb_hbm, bb_v)

    for q in range(B_PER_W // LANES):
        off = q * LANES
        urow_v[pl.ds(off, LANES)] = lax.shift_right_logical(
            uidx_v[pl.ds(off, LANES)], 7)
        crow_v[pl.ds(off, LANES)] = lax.shift_right_logical(
            cidx_v[pl.ds(off, LANES)], 7)

    def start(k, slot):
        hu = pltpu.async_copy(pu_hbm.at[urow_v.at[pl.ds(k * CH, CH)]],
                              ubufs[slot], usems[slot])
        hc = pltpu.async_copy(pc_hbm.at[crow_v.at[pl.ds(k * CH, CH)]],
                              cbufs[slot], csems[slot])
        return hu, hc

    bvec = bb_v[:]
    lane = lax.iota(jnp.int32, LANES)

    handles = [None] * NCH
    for k in range(NSLOTS):
        handles[k] = start(k, k % NSLOTS)

    for k in range(NCH):
        slot = k % NSLOTS
        hu, hc = handles[k]
        hu.wait()
        hc.wait()
        ubuf = ubufs[slot]
        cbuf = cbufs[slot]

        def gbody(g, _, ubuf=ubuf, cbuf=cbuf, k=k):
            goff = pl.multiple_of(k * CH + g * LANES, LANES)
            iu = uidx_v[pl.ds(goff, LANES)]
            ic = cidx_v[pl.ds(goff, LANES)]
            res = bvec
            for j in range(LANES):
                r = g * LANES + j

                def pick(buf, idx_vec, rb):
                    w = idx_vec[j] & 127
                    coff = pl.multiple_of(w & 112, LANES)
                    v = buf[r, pl.ds(coff, LANES)]
                    rot_v[pl.ds(rb, LANES)] = v
                    rot_v[pl.ds(rb + LANES, LANES)] = v
                    return rot_v[pl.ds(rb + (((w & 15) - j + LANES) & 15),
                                       LANES)]

                tu = pick(ubuf, iu, 4 * LANES * j)
                tc_ = pick(cbuf, ic, 4 * LANES * j + 2 * LANES)
                res = jnp.where(lane == j, res + tu + tc_, res)
            out_v[pl.ds(goff, LANES)] = res
            return 0

        lax.fori_loop(0, CGP, gbody, 0)

        nxt = k + NSLOTS
        if nxt < NCH:
            handles[nxt] = start(nxt, slot)

    pltpu.sync_copy(out_v, out_hbm.at[pl.ds(base, B_PER_W)])


def _make_sc_gather(nru, nrc):
    return functools.partial(
        pl.kernel,
        mesh=plsc.VectorSubcoreMesh(core_axis_name="c", subcore_axis_name="s"),
        out_type=jax.ShapeDtypeStruct((BATCH,), jnp.float32),
        scratch_types=[
            pltpu.VMEM((B_PER_W,), jnp.int32),
            pltpu.VMEM((B_PER_W,), jnp.int32),
            pltpu.VMEM((B_PER_W,), jnp.int32),
            pltpu.VMEM((B_PER_W,), jnp.int32),
            pltpu.VMEM((CH, 128), jnp.float32),
            pltpu.VMEM((CH, 128), jnp.float32),
            pltpu.VMEM((CH, 128), jnp.float32),
            pltpu.VMEM((CH, 128), jnp.float32),
            pltpu.VMEM((LANES,), jnp.float32),
            pltpu.VMEM((4 * LANES * LANES,), jnp.float32),
            pltpu.VMEM((B_PER_W,), jnp.float32),
            pltpu.SemaphoreType.DMA,
            pltpu.SemaphoreType.DMA,
            pltpu.SemaphoreType.DMA,
            pltpu.SemaphoreType.DMA,
        ],
    )(_sc_body)


_SC_GATHER = None


def kernel(user_ids, course_ids, user_factors, course_factors, fc_w, fc_b):
    global _SC_GATHER
    nu = user_factors.shape[0]
    ncr = course_factors.shape[0]

    wu_row = fc_w[:N_FACTORS].T
    wc_row = fc_w[N_FACTORS:].T
    p_u = _tc_project(user_factors.T, wu_row)
    p_c = _tc_project(course_factors.T, wc_row)

    nru = (nu + 127) // 128
    nrc = (ncr + 127) // 128
    pu2 = jnp.pad(p_u, (0, nru * 128 - nu)).reshape(nru, 128)
    pc2 = jnp.pad(p_c, (0, nrc * 128 - ncr)).reshape(nrc, 128)
    bb16 = jnp.broadcast_to(fc_b, (LANES,)).astype(jnp.float32)

    if _SC_GATHER is None:
        _SC_GATHER = _make_sc_gather(nru, nrc)
    return _SC_GATHER(user_ids.astype(jnp.int32),
                      course_ids.astype(jnp.int32),
                      pu2, pc2, bb16)


# speedup vs baseline: 2.8757x; 1.0355x over previous
import functools

import jax
import jax.numpy as jnp
from jax import lax
from jax.experimental import pallas as pl
from jax.experimental.pallas import tpu as pltpu
from jax.experimental.pallas import tpu_sc as plsc

N_FACTORS = 100
BATCH = 16384
LANES = 16
NC = 2
NS = 16
NW = NC * NS
B_PER_W = BATCH // NW
CH = 128
NCH = B_PER_W // CH
CGP = CH // LANES
NSLOTS = 2
MV_NB = 16384


def _mv_body(x_ref, w_ref, o_ref):
    o_ref[...] = jnp.dot(w_ref[...], x_ref[...],
                         preferred_element_type=jnp.float32)[0]


def _tc_project(xt, w_row, n_out):
    f, n = xt.shape
    grid = (pl.cdiv(n_out, MV_NB),)
    return pl.pallas_call(
        _mv_body,
        grid=grid,
        in_specs=[
            pl.BlockSpec((f, MV_NB), lambda i: (0, i)),
            pl.BlockSpec((1, f), lambda i: (0, 0)),
        ],
        out_specs=pl.BlockSpec((MV_NB,), lambda i: (i,)),
        out_shape=jax.ShapeDtypeStruct((n_out,), jnp.float32),
    )(xt, w_row)


def _sc_body(uid_hbm, cid_hbm, pu_hbm, pc_hbm, bb_hbm,
             out_hbm,
             uidx_v, cidx_v, urow_v, crow_v, ub0, ub1, cb0, cb1,
             bb_v, rot_v, out_v,
             su0, su1, sc0, sc1):
    ubufs = (ub0, ub1)
    cbufs = (cb0, cb1)
    usems = (su0, su1)
    csems = (sc0, sc1)

    wid = lax.axis_index("s") * NC + lax.axis_index("c")
    base = wid * B_PER_W

    pltpu.sync_copy(uid_hbm.at[pl.ds(base, B_PER_W)], uidx_v)
    pltpu.sync_copy(cid_hbm.at[pl.ds(base, B_PER_W)], cidx_v)
    pltpu.sync_copy(bb_hbm, bb_v)

    for q in range(B_PER_W // LANES):
        off = q * LANES
        urow_v[pl.ds(off, LANES)] = lax.shift_right_logical(
            uidx_v[pl.ds(off, LANES)], 7)
        crow_v[pl.ds(off, LANES)] = lax.shift_right_logical(
            cidx_v[pl.ds(off, LANES)], 7)

    def start(k, slot):
        hu = pltpu.async_copy(pu_hbm.at[urow_v.at[pl.ds(k * CH, CH)]],
                              ubufs[slot], usems[slot])
        hc = pltpu.async_copy(pc_hbm.at[crow_v.at[pl.ds(k * CH, CH)]],
                              cbufs[slot], csems[slot])
        return hu, hc

    bvec = bb_v[:]
    lane = lax.iota(jnp.int32, LANES)

    handles = [None] * NCH
    for k in range(NSLOTS):
        handles[k] = start(k, k % NSLOTS)

    for k in range(NCH):
        slot = k % NSLOTS
        hu, hc = handles[k]
        hu.wait()
        hc.wait()
        ubuf = ubufs[slot]
        cbuf = cbufs[slot]

        def gbody(g, _, ubuf=ubuf, cbuf=cbuf, k=k):
            goff = pl.multiple_of(k * CH + g * LANES, LANES)
            iu = uidx_v[pl.ds(goff, LANES)]
            ic = cidx_v[pl.ds(goff, LANES)]
            res = bvec
            for j in range(LANES):
                r = g * LANES + j

                def pick(buf, idx_vec, rb):
                    w = idx_vec[j] & 127
                    coff = pl.multiple_of(w & 112, LANES)
                    v = buf[r, pl.ds(coff, LANES)]
                    rot_v[pl.ds(rb, LANES)] = v
                    rot_v[pl.ds(rb + LANES, LANES)] = v
                    return rot_v[pl.ds(rb + (((w & 15) - j + LANES) & 15),
                                       LANES)]

                tu = pick(ubuf, iu, 4 * LANES * j)
                tc_ = pick(cbuf, ic, 4 * LANES * j + 2 * LANES)
                res = jnp.where(lane == j, res + tu + tc_, res)
            out_v[pl.ds(goff, LANES)] = res
            return 0

        lax.fori_loop(0, CGP, gbody, 0)

        nxt = k + NSLOTS
        if nxt < NCH:
            handles[nxt] = start(nxt, slot)

    pltpu.sync_copy(out_v, out_hbm.at[pl.ds(base, B_PER_W)])


def _make_sc_gather(nru, nrc):
    return functools.partial(
        pl.kernel,
        mesh=plsc.VectorSubcoreMesh(core_axis_name="c", subcore_axis_name="s"),
        out_type=jax.ShapeDtypeStruct((BATCH,), jnp.float32),
        scratch_types=[
            pltpu.VMEM((B_PER_W,), jnp.int32),
            pltpu.VMEM((B_PER_W,), jnp.int32),
            pltpu.VMEM((B_PER_W,), jnp.int32),
            pltpu.VMEM((B_PER_W,), jnp.int32),
            pltpu.VMEM((CH, 128), jnp.float32),
            pltpu.VMEM((CH, 128), jnp.float32),
            pltpu.VMEM((CH, 128), jnp.float32),
            pltpu.VMEM((CH, 128), jnp.float32),
            pltpu.VMEM((LANES,), jnp.float32),
            pltpu.VMEM((4 * LANES * LANES,), jnp.float32),
            pltpu.VMEM((B_PER_W,), jnp.float32),
            pltpu.SemaphoreType.DMA,
            pltpu.SemaphoreType.DMA,
            pltpu.SemaphoreType.DMA,
            pltpu.SemaphoreType.DMA,
        ],
    )(_sc_body)


_SC_GATHER = None


def kernel(user_ids, course_ids, user_factors, course_factors, fc_w, fc_b):
    global _SC_GATHER
    nu = user_factors.shape[0]
    ncr = course_factors.shape[0]

    wu_row = fc_w[:N_FACTORS].T
    wc_row = fc_w[N_FACTORS:].T
    nru = -(-nu // 1024) * 8
    nrc = -(-ncr // 1024) * 8
    p_u = _tc_project(user_factors.T, wu_row, nru * 128)
    p_c = _tc_project(course_factors.T, wc_row, nrc * 128)
    pu2 = p_u.reshape(nru, 128)
    pc2 = p_c.reshape(nrc, 128)
    bb16 = jnp.broadcast_to(fc_b, (LANES,)).astype(jnp.float32)

    if _SC_GATHER is None:
        _SC_GATHER = _make_sc_gather(nru, nrc)
    return _SC_GATHER(user_ids.astype(jnp.int32),
                      course_ids.astype(jnp.int32),
                      pu2, pc2, bb16)
